# baseline (speedup 1.0000x reference)
import functools

import jax
import jax.numpy as jnp
from jax import lax
from jax.experimental import pallas as pl
from jax.experimental.pallas import tpu as pltpu
from jax.experimental.pallas import tpu_sc as plsc

N = 10000
R = 4
RN = R * N
E = 320000
EMB = 128

NC = 2
NS = 16
NW = NC * NS

EPAD = 327680
NROWS = EPAD // 128
PER_W = NROWS // NW
B0 = 16
B1 = 20 - B0
PER_T = NROWS // NS
BLK = 8
SUB = 4 * BLK
NSLOT = 8
DEPTH = 6
ACC = 10112
ROWS_T = ACC // NS
DEGSZ = 40960
DEG_T = DEGSZ // NS
ZB = 320

BN = 1000


def _bmm_body(x_ref, w_ref, o_ref):
    o_ref[...] = jnp.dot(x_ref[...], w_ref[0],
                         preferred_element_type=jnp.float32)


def _tc_bmm(x, weights):
    return pl.pallas_call(
        _bmm_body,
        grid=(R, N // BN),
        in_specs=[
            pl.BlockSpec((BN, EMB), lambda r, i: (i, 0)),
            pl.BlockSpec((1, EMB, EMB), lambda r, i: (r, 0, 0)),
        ],
        out_specs=pl.BlockSpec((BN, EMB), lambda r, i: (r * (N // BN) + i, 0)),
        out_shape=jax.ShapeDtypeStruct((RN, EMB), jnp.float32),
    )(x, weights)


def _fin_body(p_ref, o_ref):
    o_ref[...] = jnp.maximum(p_ref[0] + p_ref[1], 0.0)


def _tc_finish(partial):
    return pl.pallas_call(
        _fin_body,
        grid=(N // BN,),
        in_specs=[pl.BlockSpec((2, BN, EMB), lambda i: (0, i, 0))],
        out_specs=pl.BlockSpec((BN, EMB), lambda i: (i, 0)),
        out_shape=jax.ShapeDtypeStruct((N, EMB), jnp.float32),
    )(partial)


def _sc_body(y_hbm, er_hbm, ec_hbm, zacc_hbm, out_hbm,
             acc_sh, deg_sh, erbuf, ecbuf, valbuf, dstbuf, ones, zbuf, rb,
             gsem, ssem):
    c = lax.axis_index("c")
    s = lax.axis_index("s")
    w = c * NS + s

    def zfill(j, _):
        zbuf[pl.ds(j * 16, 16)] = jnp.zeros((16,), jnp.float32)
        return 0

    lax.fori_loop(0, ZB // 16, zfill, 0)
    pltpu.sync_copy(zacc_hbm.at[pl.ds(s * ROWS_T, ROWS_T)],
                    acc_sh.at[pl.ds(s * ROWS_T, ROWS_T)])
    for z in range(DEG_T // ZB):
        pltpu.sync_copy(zbuf, deg_sh.at[pl.ds(s * DEG_T + z * ZB, ZB)])
    for b in range(8):
        ones[pl.ds(b * 16, 16)] = jnp.full((16,), 1.0, jnp.float32)
    plsc.subcore_barrier()

    def deg_blk(blk, _):
        pltpu.sync_copy(er_hbm.at[pl.ds(s * PER_T + blk * BLK, BLK)],
                        erbuf.at[0])
        descs = [pltpu.async_copy(ones, deg_sh.at[erbuf.at[0, j]], gsem,
                                  add=True)
                 for j in range(BLK)]
        for d in descs:
            d.wait()
        return 0

    lax.fori_loop(0, PER_T // BLK, deg_blk, 0)
    plsc.subcore_barrier()

    nblk = jnp.where(c == 0, B0, B1)
    wbase = jnp.where(c == 0, s * (B0 * BLK), NS * (B0 * BLK) + s * (B1 * BLK))

    def prep(pi, base):
        descs = [pltpu.async_copy(deg_sh.at[erbuf.at[pi, j]],
                                  valbuf.at[pi, j], ssem)
                 for j in range(BLK)]
        for d in descs:
            d.wait()

        def vcompute(j, _):
            for b in range(8):
                sl = pl.ds(b * 16, 16)
                er_v = erbuf[pi, j, sl]
                ec_v = ecbuf[pi, j, sl]
                dv = valbuf[pi, j, sl]
                dstm = lax.rem(er_v, N)
                pad = er_v >= RN
                gi = jnp.where(pad, 0, er_v - dstm + ec_v)
                dstbuf[j * 4 + b // 2, pl.ds((b % 2) * 16, 16)] = (
                    jnp.where(pad, ACC - 16, dstm))
                ecbuf[pi, j, sl] = gi
                valbuf[pi, j, sl] = jnp.where(pad, 0.0, 1.0 / dv)
            return 0

        lax.fori_loop(0, BLK, vcompute, 0)

    pltpu.sync_copy(er_hbm.at[pl.ds(wbase, BLK)], erbuf.at[0])
    pltpu.sync_copy(ec_hbm.at[pl.ds(wbase, BLK)], ecbuf.at[0])
    prep(0, wbase)

    def block(bk, _):
        p = lax.rem(bk, 2)
        np_ = 1 - p
        nbase = jnp.minimum(wbase + (bk + 1) * BLK, NROWS - BLK)
        sd = [pltpu.async_copy(er_hbm.at[pl.ds(nbase, BLK)], erbuf.at[np_],
                               ssem),
              pltpu.async_copy(ec_hbm.at[pl.ds(nbase, BLK)], ecbuf.at[np_],
                               ssem)]

        def gather_k(k):
            jc, t = k // 4, k % 4
            return pltpu.async_copy(
                y_hbm.at[ecbuf.at[p, jc, pl.ds(32 * t, 32)]],
                rb.at[k % NSLOT], gsem)

        descs = [gather_k(k) for k in range(DEPTH)]
        for k in range(SUB):
            if k + DEPTH < SUB:
                descs.append(gather_k(k + DEPTH))
            descs[k].wait()
            jc, t, slot = k // 4, k % 4, k % NSLOT

            def rowfn(ju, _, jc=jc, t=t, slot=slot):
                for u in range(4):
                    jj = ju * 4 + u
                    spl = plsc.load_gather(
                        valbuf, [jnp.full((16,), p, jnp.int32),
                                 jnp.full((16,), jc, jnp.int32),
                                 jnp.full((16,), 32 * t, jnp.int32) + jj])
                    for b in range(EMB // 16):
                        sl = pl.ds(b * 16, 16)
                        rb[slot, jj, sl] = rb[slot, jj, sl] * spl
                return 0

            lax.fori_loop(0, 8, rowfn, 0)
            pltpu.sync_copy(rb.at[slot], acc_sh.at[dstbuf.at[k]], add=True)

        for d in sd:
            d.wait()
        prep(np_, nbase)
        return 0

    lax.fori_loop(0, nblk, block, 0)
    plsc.subcore_barrier()

    pltpu.sync_copy(acc_sh.at[pl.ds(s * ROWS_T, ROWS_T)],
                    out_hbm.at[c, pl.ds(s * ROWS_T, ROWS_T)])


_sc_main = functools.partial(
    pl.kernel,
    out_type=jax.ShapeDtypeStruct((NC, ACC, EMB), jnp.float32),
    mesh=plsc.VectorSubcoreMesh(core_axis_name="c", subcore_axis_name="s",
                                num_cores=NC, num_subcores=NS),
    compiler_params=pltpu.CompilerParams(needs_layout_passes=False),
    scratch_types=[
        pltpu.VMEM_SHARED((ACC, EMB), jnp.float32),
        pltpu.VMEM_SHARED((DEGSZ,), jnp.float32),
        pltpu.VMEM((2, BLK, 128), jnp.int32),
        pltpu.VMEM((2, BLK, 128), jnp.int32),
        pltpu.VMEM((2, BLK, 128), jnp.float32),
        pltpu.VMEM((SUB, 32), jnp.int32),
        pltpu.VMEM((128,), jnp.float32),
        pltpu.VMEM((ZB,), jnp.float32),
        pltpu.VMEM((NSLOT, 32, EMB), jnp.float32),
        pltpu.SemaphoreType.DMA,
        pltpu.SemaphoreType.DMA,
    ],
)(_sc_body)


def kernel(x, edge_rows, edge_cols, weights):
    y2 = _tc_bmm(x, weights)

    pad = EPAD - E
    er = jnp.concatenate(
        [edge_rows, jnp.full((pad,), RN, jnp.int32)]).reshape(NROWS, 128)
    ec = jnp.concatenate(
        [edge_cols, jnp.zeros((pad,), jnp.int32)]).reshape(NROWS, 128)

    zacc = jnp.zeros((ACC, EMB), jnp.float32)

    partial = _sc_main(y2, er, ec, zacc)
    return _tc_finish(partial)


# speedup vs baseline: 1.0092x; 1.0092x over previous
import functools

import jax
import jax.numpy as jnp
from jax import lax
from jax.experimental import pallas as pl
from jax.experimental.pallas import tpu as pltpu
from jax.experimental.pallas import tpu_sc as plsc

N = 10000
R = 4
RN = R * N
E = 320000
EMB = 128

NC = 2
NS = 16
NW = NC * NS

EPAD = 327680
NROWS = EPAD // 128
PER_W = NROWS // NW
B0 = 16
B1 = 20 - B0
PER_T = NROWS // NS
BLK = 8
SUB = 4 * BLK
NSLOT = 8
DEPTH = 6
ACC = 10112
ROWS_T = ACC // NS
DEGSZ = 40960
DEG_T = DEGSZ // NS
ZB = 320

BN = 1000


def _bmm_body(x_ref, w_ref, o_ref):
    o_ref[...] = jnp.dot(x_ref[...], w_ref[0],
                         preferred_element_type=jnp.float32)


def _tc_bmm(x, weights):
    return pl.pallas_call(
        _bmm_body,
        grid=(R, N // BN),
        in_specs=[
            pl.BlockSpec((BN, EMB), lambda r, i: (i, 0)),
            pl.BlockSpec((1, EMB, EMB), lambda r, i: (r, 0, 0)),
        ],
        out_specs=pl.BlockSpec((BN, EMB), lambda r, i: (r * (N // BN) + i, 0)),
        out_shape=jax.ShapeDtypeStruct((RN, EMB), jnp.float32),
    )(x, weights)


def _fin_body(p_ref, o_ref):
    o_ref[...] = jnp.maximum(p_ref[0] + p_ref[1], 0.0)


def _tc_finish(partial):
    return pl.pallas_call(
        _fin_body,
        grid=(N // BN,),
        in_specs=[pl.BlockSpec((2, BN, EMB), lambda i: (0, i, 0))],
        out_specs=pl.BlockSpec((BN, EMB), lambda i: (i, 0)),
        out_shape=jax.ShapeDtypeStruct((N, EMB), jnp.float32),
    )(partial)


def _sc_body(y_hbm, er_hbm, ec_hbm, zacc_hbm, out_hbm,
             acc_sh, deg_sh, erbuf, ecbuf, valbuf, dstbuf, ones, zbuf, rb,
             gsem):
    c = lax.axis_index("c")
    s = lax.axis_index("s")
    w = c * NS + s

    def zfill(j, _):
        zbuf[pl.ds(j * 16, 16)] = jnp.zeros((16,), jnp.float32)
        return 0

    lax.fori_loop(0, ZB // 16, zfill, 0)
    pltpu.sync_copy(zacc_hbm.at[pl.ds(s * ROWS_T, ROWS_T)],
                    acc_sh.at[pl.ds(s * ROWS_T, ROWS_T)])
    for z in range(DEG_T // ZB):
        pltpu.sync_copy(zbuf, deg_sh.at[pl.ds(s * DEG_T + z * ZB, ZB)])
    for b in range(8):
        ones[pl.ds(b * 16, 16)] = jnp.full((16,), 1.0, jnp.float32)
    plsc.subcore_barrier()

    def deg_blk(blk, _):
        pltpu.sync_copy(er_hbm.at[pl.ds(s * PER_T + blk * BLK, BLK)], erbuf)
        descs = [pltpu.async_copy(ones, deg_sh.at[erbuf.at[j]], gsem,
                                  add=True)
                 for j in range(BLK)]
        for d in descs:
            d.wait()
        return 0

    lax.fori_loop(0, PER_T // BLK, deg_blk, 0)
    plsc.subcore_barrier()

    nblk = jnp.where(c == 0, B0, B1)
    wbase = jnp.where(c == 0, s * (B0 * BLK), NS * (B0 * BLK) + s * (B1 * BLK))

    def block(bk, _):
        base = wbase + bk * BLK
        pltpu.sync_copy(er_hbm.at[pl.ds(base, BLK)], erbuf)
        pltpu.sync_copy(ec_hbm.at[pl.ds(base, BLK)], ecbuf)

        descs = [pltpu.async_copy(deg_sh.at[erbuf.at[j]], valbuf.at[j], gsem)
                 for j in range(BLK)]
        for d in descs:
            d.wait()

        def vcompute(j, _):
            for b in range(8):
                sl = pl.ds(b * 16, 16)
                er_v = erbuf[j, sl]
                ec_v = ecbuf[j, sl]
                dv = valbuf[j, sl]
                dstm = lax.rem(er_v, N)
                pad = er_v >= RN
                gi = jnp.where(pad, 0, er_v - dstm + ec_v)
                dstbuf[j * 4 + b // 2, pl.ds((b % 2) * 16, 16)] = (
                    jnp.where(pad, ACC - 16, dstm))
                ecbuf[j, sl] = gi
                valbuf[j, sl] = jnp.where(pad, 0.0, 1.0 / dv)
            return 0

        lax.fori_loop(0, BLK, vcompute, 0)

        def gather_k(k):
            jc, t = k // 4, k % 4
            return pltpu.async_copy(
                y_hbm.at[ecbuf.at[jc, pl.ds(32 * t, 32)]],
                rb.at[k % NSLOT], gsem)

        descs = [gather_k(k) for k in range(DEPTH)]
        for k in range(SUB):
            if k + DEPTH < SUB:
                descs.append(gather_k(k + DEPTH))
            descs[k].wait()
            jc, t, slot = k // 4, k % 4, k % NSLOT

            def rowfn(ju, _, jc=jc, t=t, slot=slot):
                for u in range(4):
                    jj = ju * 4 + u
                    spl = plsc.load_gather(
                        valbuf, [jnp.full((16,), jc, jnp.int32),
                                 jnp.full((16,), 32 * t, jnp.int32) + jj])
                    for b in range(EMB // 16):
                        sl = pl.ds(b * 16, 16)
                        rb[slot, jj, sl] = rb[slot, jj, sl] * spl
                return 0

            lax.fori_loop(0, 8, rowfn, 0)
            pltpu.sync_copy(rb.at[slot], acc_sh.at[dstbuf.at[k]], add=True)
        return 0

    lax.fori_loop(0, nblk, block, 0)
    plsc.subcore_barrier()

    pltpu.sync_copy(acc_sh.at[pl.ds(s * ROWS_T, ROWS_T)],
                    out_hbm.at[c, pl.ds(s * ROWS_T, ROWS_T)])


_sc_main = functools.partial(
    pl.kernel,
    out_type=jax.ShapeDtypeStruct((NC, ACC, EMB), jnp.float32),
    mesh=plsc.VectorSubcoreMesh(core_axis_name="c", subcore_axis_name="s",
                                num_cores=NC, num_subcores=NS),
    compiler_params=pltpu.CompilerParams(needs_layout_passes=False),
    scratch_types=[
        pltpu.VMEM_SHARED((ACC, EMB), jnp.float32),
        pltpu.VMEM_SHARED((DEGSZ,), jnp.float32),
        pltpu.VMEM((BLK, 128), jnp.int32),
        pltpu.VMEM((BLK, 128), jnp.int32),
        pltpu.VMEM((BLK, 128), jnp.float32),
        pltpu.VMEM((SUB, 32), jnp.int32),
        pltpu.VMEM((128,), jnp.float32),
        pltpu.VMEM((ZB,), jnp.float32),
        pltpu.VMEM((NSLOT, 32, EMB), jnp.float32),
        pltpu.SemaphoreType.DMA,
    ],
)(_sc_body)


def kernel(x, edge_rows, edge_cols, weights):
    y2 = _tc_bmm(x, weights)

    pad = EPAD - E
    er = jnp.concatenate(
        [edge_rows, jnp.full((pad,), RN, jnp.int32)]).reshape(NROWS, 128)
    ec = jnp.concatenate(
        [edge_cols, jnp.zeros((pad,), jnp.int32)]).reshape(NROWS, 128)

    zacc = jnp.zeros((ACC, EMB), jnp.float32)

    partial = _sc_main(y2, er, ec, zacc)
    return _tc_finish(partial)
